# Initial kernel scaffold; baseline (speedup 1.0000x reference)
import jax
import jax.numpy as jnp
from jax.experimental import pallas as pl


def kernel(input_ids, emb):
    raise NotImplementedError("write your pallas kernel here")


# speedup vs baseline: 1.1389x; 1.1389x over previous
import functools

import jax
import jax.numpy as jnp
from jax import lax
from jax.experimental import pallas as pl
from jax.experimental.pallas import tpu as pltpu
from jax.experimental.pallas import tpu_sc as plsc

_EPS = 4e-3
_MIN_NORM = 1e-15
_CH = 100


def _logmap_scale(x):
    ss = jnp.sum(x * x, axis=-1, keepdims=True)
    norm = jnp.maximum(jnp.sqrt(ss), _MIN_NORM)
    arg = jnp.minimum(norm, 1.0 - 1e-7)
    return 0.5 * jnp.log((1.0 + arg) / (1.0 - arg)) / norm


def _logmap_table_body(x_ref, out_ref):
    x = x_ref[...]
    t = x * _logmap_scale(x)
    out_ref[:, 0:64] = t
    out_ref[:, 64:128] = t


def _logmap_table(emb):
    v, d = emb.shape
    blk = 8000
    nblk = v // blk
    assert v % blk == 0 and d == 64
    return pl.pallas_call(
        _logmap_table_body,
        grid=(nblk,),
        in_specs=[pl.BlockSpec((blk, d), lambda i: (i, 0))],
        out_specs=pl.BlockSpec((blk, 2 * d), lambda i: (i, 0)),
        out_shape=jax.ShapeDtypeStruct((v, 2 * d), jnp.float32),
    )(emb)


def _finalize_body(t_tokens, sum_ref, out_ref):
    u = sum_ref[...] * (1.0 / t_tokens)
    ss = jnp.sum(u * u, axis=-1, keepdims=True)
    norm = jnp.maximum(jnp.sqrt(ss), _MIN_NORM)
    y = jnp.tanh(norm) * u / norm
    ssy = jnp.sum(y * y, axis=-1, keepdims=True)
    ny = jnp.maximum(jnp.sqrt(ssy), _MIN_NORM)
    maxn = 1.0 - _EPS
    out_ref[...] = jnp.where(ny > maxn, y / ny * maxn, y)


def _finalize(sums, t_tokens):
    b, d = sums.shape
    return pl.pallas_call(
        functools.partial(_finalize_body, float(t_tokens)),
        out_shape=jax.ShapeDtypeStruct((b, d), jnp.float32),
    )(sums)


def _sc_gather_sum(tang2, ids2, b, t_tokens, d):
    info = plsc.get_sparse_core_info()
    nw = info.num_cores * info.num_subcores
    assert b % nw == 0 and t_tokens % _CH == 0 and d == 64
    sper = b // nw
    cps = t_tokens // _CH
    cpw = sper * cps
    assert cps == 2

    mesh = plsc.VectorSubcoreMesh(core_axis_name="c", subcore_axis_name="s")

    @functools.partial(
        pl.kernel,
        out_type=jax.ShapeDtypeStruct((b, d), jnp.float32),
        mesh=mesh,
        scratch_types=[
            pltpu.VMEM((cpw, _CH), jnp.int32),
            pltpu.VMEM((2, _CH, 2 * d), jnp.float32),
            pltpu.VMEM((sper, d), jnp.float32),
            pltpu.SemaphoreType.DMA,
            pltpu.SemaphoreType.DMA,
        ],
    )
    def k(tang_hbm, idx_hbm, out_hbm, idx_v, rows_v, out_v, sem0, sem1):
        wid = lax.axis_index("s") * info.num_cores + lax.axis_index("c")
        base_chunk = wid * cpw
        pltpu.sync_copy(idx_hbm.at[pl.ds(base_chunk, cpw)], idx_v)
        sems = (sem0, sem1)
        for bslot in range(2):
            pltpu.async_copy(
                tang_hbm.at[idx_v.at[bslot]], rows_v.at[bslot], sems[bslot]
            )

        def sentence(s, _):
            zero = jnp.zeros((16,), jnp.float32)
            acc = (zero, zero, zero, zero)
            for bslot in range(2):
                c = 2 * s + bslot
                pltpu.make_async_copy(
                    tang_hbm.at[idx_v.at[c]], rows_v.at[bslot], sems[bslot]
                ).wait()

                def body(tok, a, bslot=bslot):
                    return tuple(
                        a[k] + rows_v[bslot, tok, pl.ds(k * 16, 16)]
                        for k in range(4)
                    )

                acc = lax.fori_loop(0, _CH, body, acc, unroll=4)

                @pl.when(s < sper - 1)
                def _(bslot=bslot, c=c):
                    pltpu.async_copy(
                        tang_hbm.at[idx_v.at[c + 2]],
                        rows_v.at[bslot],
                        sems[bslot],
                    )
            for k4 in range(4):
                out_v[s, pl.ds(k4 * 16, 16)] = acc[k4]
            return 0

        lax.fori_loop(0, sper, sentence, 0)
        pltpu.sync_copy(out_v, out_hbm.at[pl.ds(wid * sper, sper)])

    return k(tang2, ids2)


def kernel(input_ids, emb):
    b, t_tokens = input_ids.shape
    v, d = emb.shape
    tang2 = _logmap_table(emb)
    ids2 = input_ids.reshape(b * t_tokens // _CH, _CH)
    sums = _sc_gather_sum(tang2, ids2, b, t_tokens, d)
    return _finalize(sums, t_tokens)
